# baseline (speedup 1.0000x reference)
import jax
import jax.numpy as jnp
from jax import lax
from jax.experimental import pallas as pl
from jax.experimental.pallas import tpu as pltpu
from jax.experimental.pallas import tpu_sc as plsc

N = 100000
R = 8
D = 128
ROW = R * D
ROWP = ROW + 1
OUTP = D + 1
NC = 2
NS = 16
NW = NC * NS
L = 16
CH = 32
NCHUNK = N // CH
NI = (NCHUNK + NW - 1) // NW
NPAIR = NI // 2


def _sc_body(x_hbm, out_hbm, in0, in1, ou0, ou1, si0, si1, so0, so1):
    wid = lax.axis_index("s") * NC + lax.axis_index("c")
    lanes = lax.broadcasted_iota(jnp.int32, (L,), 0)
    in_bufs, out_bufs = (in0, in1), (ou0, ou1)
    in_sems, out_sems = (si0, si1), (so0, so1)

    def start_row(i):
        return jnp.minimum(wid + i * NW, NCHUNK - 1) * CH

    def in_copy(i, b):
        return pltpu.make_async_copy(
            x_hbm.at[pl.ds(start_row(i), CH)],
            in_bufs[b].at[:, pl.ds(0, ROW)], in_sems[b])

    def out_copy(i, b):
        return pltpu.make_async_copy(
            out_bufs[b].at[:, pl.ds(0, D)],
            out_hbm.at[pl.ds(start_row(i), CH)], out_sems[b])

    def compute(b):
        in_v, out_v = in_bufs[b], out_bufs[b]
        for g in range(CH // L):
            rowv = lanes + g * L

            def norm_body(f, st):
                col = st[0]
                accs = list(st[1:])
                for r in range(R):
                    v = plsc.load_gather(in_v, [rowv, col + (r * D)])
                    accs[r] = accs[r] + v * v
                return (col + 1,) + tuple(accs)

            zero_i = jnp.zeros((L,), jnp.int32)
            zero_f = jnp.zeros((L,), jnp.float32)
            st = lax.fori_loop(0, D, norm_body,
                               (zero_i,) + (zero_f,) * R, unroll=4)
            accs = st[1:]
            best = jnp.zeros((L,), jnp.int32)
            bestv = accs[0]
            for r in range(1, R):
                m = accs[r] > bestv
                bestv = jnp.where(m, accs[r], bestv)
                best = jnp.where(m, jnp.full((L,), r, jnp.int32), best)

            def copy_body(f, st):
                col, ocol = st
                v = plsc.load_gather(in_v, [rowv, col])
                plsc.store_scatter(out_v, [rowv, ocol], v)
                return (col + 1, ocol + 1)

            lax.fori_loop(0, D, copy_body, (best * D, zero_i), unroll=4)

    in_copy(0, 0).start()
    in_copy(1, 1).start()
    in_copy(0, 0).wait()
    compute(0)
    out_copy(0, 0).start()
    in_copy(2, 0).start()
    in_copy(1, 1).wait()
    compute(1)
    out_copy(1, 1).start()

    def pair_body(j, carry):
        for b in range(2):
            i = 2 * j + b
            in_copy(i + 1, 1 - b).start()
            in_copy(i, b).wait()
            out_copy(i - 2, b).wait()
            compute(b)
            out_copy(i, b).start()
        return carry

    lax.fori_loop(1, NPAIR, pair_body, 0)

    in_copy(NI, 0).wait()
    out_copy(NI - 2, 0).wait()
    out_copy(NI - 1, 1).wait()


def kernel(inputs):
    x = inputs.reshape(N, ROW)
    mesh = plsc.VectorSubcoreMesh(core_axis_name="c", subcore_axis_name="s")
    f = pl.kernel(
        _sc_body, mesh=mesh,
        out_type=jax.ShapeDtypeStruct((N, D), jnp.float32),
        scratch_types=[
            pltpu.VMEM((CH, ROWP), jnp.float32),
            pltpu.VMEM((CH, ROWP), jnp.float32),
            pltpu.VMEM((CH, OUTP), jnp.float32),
            pltpu.VMEM((CH, OUTP), jnp.float32),
            pltpu.SemaphoreType.DMA,
            pltpu.SemaphoreType.DMA,
            pltpu.SemaphoreType.DMA,
            pltpu.SemaphoreType.DMA,
        ],
        compiler_params=pltpu.CompilerParams(needs_layout_passes=False),
    )
    return f(x)
